# baseline (speedup 1.0000x reference)
import jax
import jax.numpy as jnp
from jax import lax
from jax.experimental import pallas as pl
from jax.experimental.pallas import tpu as pltpu
from jax.experimental.pallas import tpu_sc as plsc

B = 16384
D = 65
DP = 128
V = 1000000
H = 50
HH = 25
NC = 2
NS = 16
NW = NC * NS
R = B // NW
CS = 128
NCH = R // CS
NBUF = 2
IDEPTH = 8


def _fedrec_body(items_hbm, ftab_hbm, iids_hbm, fids_hbm, out_hbm,
                 iidx_v, fidx_v, ramp_v, feat_v, fli_v, icol_v, out_v,
                 acc_sh, sem, sem2):
    c = lax.axis_index("c")
    s = lax.axis_index("s")
    wid = s * NC + c
    blk = wid * NCH
    iota = lax.iota(jnp.int32, 16)

    pltpu.sync_copy(iids_hbm.at[pl.ds(blk, NCH)], iidx_v)

    for sc in range(NCH):
        for k in range(CS // 16):
            ramp_v[sc, pl.ds(k * 16, 16)] = (
                s * R + sc * CS + k * 16 + iota
            )

    zbuf = feat_v.at[0]

    def zero_row(r, carry):
        z16 = jnp.zeros((16,), jnp.float32)
        for j in range(DP // 16):
            zbuf[r, pl.ds(j * 16, 16)] = z16
        return carry

    lax.fori_loop(0, CS, zero_row, 0)
    for sc in range(NCH):
        pltpu.sync_copy(zbuf, acc_sh.at[pl.ds(s * R + sc * CS, CS), :])

    def pool_step(h, carry):
        for sc0 in range(0, NCH, NBUF):
            g = [
                pltpu.async_copy(
                    ftab_hbm.at[fidx_v.at[h, sc0 + i]],
                    feat_v.at[i],
                    sem,
                )
                for i in range(NBUF)
            ]
            for d in g:
                d.wait()
            sca = [
                pltpu.async_copy(
                    feat_v.at[i],
                    acc_sh.at[ramp_v.at[sc0 + i]],
                    sem,
                    add=True,
                )
                for i in range(NBUF)
            ]
            for d in sca:
                d.wait()
        return carry

    for hh in range(H // HH):
        pltpu.sync_copy(
            fids_hbm.at[pl.ds(hh * HH, HH), pl.ds(blk, NCH), :], fidx_v
        )
        lax.fori_loop(0, HH, pool_step, 0)

    pref_c = feat_v.at[1]
    for sc in range(NCH):
        def fli_row(d, carry):
            for k in range(CS // 16):
                fli_v[d, pl.ds(k * 16, 16)] = (
                    d * V + iidx_v[sc, pl.ds(k * 16, 16)]
                )
            return carry

        lax.fori_loop(0, D, fli_row, 0)

        rb = pltpu.async_copy(
            acc_sh.at[pl.ds(s * R + sc * CS, CS), :], pref_c, sem
        )
        for d in range(IDEPTH):
            pltpu.async_copy(
                items_hbm.at[fli_v.at[d]], icol_v.at[d], sem2
            )

        def ig_body(d, carry):
            pltpu.async_copy(items_hbm.at[fli_v.at[d]], icol_v.at[d], sem2)
            pltpu.make_async_copy(
                items_hbm.at[fli_v.at[0]], icol_v.at[0], sem2
            ).wait()
            return carry

        lax.fori_loop(IDEPTH, D, ig_body, 0)
        for _ in range(IDEPTH):
            pltpu.make_async_copy(
                items_hbm.at[fli_v.at[0]], icol_v.at[0], sem2
            ).wait()
        rb.wait()

        def grp_body(g, carry):
            rows16 = g * 16 + iota

            def d_body(d, acc):
                a = icol_v[d, pl.ds(g * 16, 16)]
                b = plsc.load_gather(
                    pref_c, [rows16, jnp.full((16,), 0, jnp.int32) + d]
                )
                return acc + a * b

            acc = lax.fori_loop(0, D, d_body, jnp.zeros((16,), jnp.float32))
            out_v[pl.ds(sc * CS + g * 16, 16)] = acc
            return carry

        lax.fori_loop(0, CS // 16, grp_body, 0)

    pltpu.sync_copy(out_v, out_hbm.at[pl.ds(wid * R, R)])


@jax.jit
def kernel(items_emb, feature_emb, item_ids, feature_ids):
    items_f = items_emb.T.reshape(-1)
    ftab_p = jnp.pad(feature_emb, ((0, 0), (0, DP - D)))
    iids = item_ids.astype(jnp.int32).reshape(B // CS, CS)
    fids = feature_ids.astype(jnp.int32).T.reshape(H, B // CS, CS)
    mesh = plsc.VectorSubcoreMesh(core_axis_name="c", subcore_axis_name="s")
    run = pl.kernel(
        _fedrec_body,
        out_type=jax.ShapeDtypeStruct((B,), jnp.float32),
        mesh=mesh,
        scratch_types=[
            pltpu.VMEM((NCH, CS), jnp.int32),
            pltpu.VMEM((HH, NCH, CS), jnp.int32),
            pltpu.VMEM((NCH, CS), jnp.int32),
            pltpu.VMEM((NBUF, CS, DP), jnp.float32),
            pltpu.VMEM((D, CS), jnp.int32),
            pltpu.VMEM((D, CS), jnp.float32),
            pltpu.VMEM((R,), jnp.float32),
            pltpu.VMEM_SHARED((NS * R, DP), jnp.float32),
            pltpu.SemaphoreType.DMA,
            pltpu.SemaphoreType.DMA,
        ],
        compiler_params=pltpu.CompilerParams(
            needs_layout_passes=False, use_tc_tiling_on_sc=False
        ),
    )
    return run(items_f, ftab_p, iids, fids)


# speedup vs baseline: 2.6709x; 2.6709x over previous
import jax
import jax.numpy as jnp
from jax import lax
from jax.experimental import pallas as pl
from jax.experimental.pallas import tpu as pltpu
from jax.experimental.pallas import tpu_sc as plsc

B = 16384
D = 65
DP = 128
V = 1000000
H = 50
HH = 25
NC = 2
NS = 16
NW = NC * NS
R = B // NW
CS = 128
NCH = R // CS
NBUF = 2
IDEPTH = 8


def _fedrec_body(items_hbm, ftab_hbm, iids_hbm, fids_hbm, out_hbm,
                 iidx_v, fidx_v, ramp_v, feat_v, fli_v, icol_v, out_v,
                 acc_sh, sem, sem2):
    c = lax.axis_index("c")
    s = lax.axis_index("s")
    wid = s * NC + c
    blk = wid * NCH
    iota = lax.iota(jnp.int32, 16)

    pltpu.sync_copy(iids_hbm.at[pl.ds(blk, NCH)], iidx_v)

    for sc in range(NCH):
        for k in range(CS // 16):
            ramp_v[sc, pl.ds(k * 16, 16)] = (
                s * R + sc * CS + k * 16 + iota
            )

    zbuf = feat_v.at[0]

    def zero_row(r, carry):
        z16 = jnp.zeros((16,), jnp.float32)
        for j in range(DP // 16):
            zbuf[r, pl.ds(j * 16, 16)] = z16
        return carry

    lax.fori_loop(0, CS, zero_row, 0)
    for sc in range(NCH):
        pltpu.sync_copy(zbuf, acc_sh.at[pl.ds(s * R + sc * CS, CS), :])

    def pool_step(h, carry):
        for sc0 in range(0, NCH, NBUF):
            g = [
                pltpu.async_copy(
                    ftab_hbm.at[fidx_v.at[h, sc0 + i]],
                    feat_v.at[i],
                    sem,
                )
                for i in range(NBUF)
            ]
            for d in g:
                d.wait()
            sca = [
                pltpu.async_copy(
                    feat_v.at[i],
                    acc_sh.at[ramp_v.at[sc0 + i]],
                    sem,
                    add=True,
                )
                for i in range(NBUF)
            ]
            for d in sca:
                d.wait()
        return carry

    for hh in range(H // HH):
        pltpu.sync_copy(
            fids_hbm.at[pl.ds(hh * HH, HH), pl.ds(blk, NCH), :], fidx_v
        )
        lax.fori_loop(0, HH, pool_step, 0)

    pref_c = feat_v.at[1]
    for sc in range(NCH):
        def fli_row(d, carry):
            for k in range(CS // 16):
                fli_v[d, pl.ds(k * 16, 16)] = (
                    iidx_v[sc, pl.ds(k * 16, 16)] * D + d
                )
            return carry

        lax.fori_loop(0, D, fli_row, 0)

        rb = pltpu.async_copy(
            acc_sh.at[pl.ds(s * R + sc * CS, CS), :], pref_c, sem
        )
        for d in range(IDEPTH):
            pltpu.async_copy(
                items_hbm.at[fli_v.at[d]], icol_v.at[d], sem2
            )

        def ig_body(d, carry):
            pltpu.async_copy(items_hbm.at[fli_v.at[d]], icol_v.at[d], sem2)
            pltpu.make_async_copy(
                items_hbm.at[fli_v.at[0]], icol_v.at[0], sem2
            ).wait()
            return carry

        lax.fori_loop(IDEPTH, D, ig_body, 0)
        for _ in range(IDEPTH):
            pltpu.make_async_copy(
                items_hbm.at[fli_v.at[0]], icol_v.at[0], sem2
            ).wait()
        rb.wait()

        def grp_body(g, carry):
            rows16 = g * 16 + iota

            def d_body(d, acc):
                a = icol_v[d, pl.ds(g * 16, 16)]
                b = plsc.load_gather(
                    pref_c, [rows16, jnp.full((16,), 0, jnp.int32) + d]
                )
                return acc + a * b

            acc = lax.fori_loop(0, D, d_body, jnp.zeros((16,), jnp.float32))
            out_v[pl.ds(sc * CS + g * 16, 16)] = acc
            return carry

        lax.fori_loop(0, CS // 16, grp_body, 0)

    pltpu.sync_copy(out_v, out_hbm.at[pl.ds(wid * R, R)])


@jax.jit
def kernel(items_emb, feature_emb, item_ids, feature_ids):
    items_f = items_emb.reshape(-1)
    ftab_p = jnp.pad(feature_emb, ((0, 0), (0, DP - D)))
    iids = item_ids.astype(jnp.int32).reshape(B // CS, CS)
    fids = feature_ids.astype(jnp.int32).T.reshape(H, B // CS, CS)
    mesh = plsc.VectorSubcoreMesh(core_axis_name="c", subcore_axis_name="s")
    run = pl.kernel(
        _fedrec_body,
        out_type=jax.ShapeDtypeStruct((B,), jnp.float32),
        mesh=mesh,
        scratch_types=[
            pltpu.VMEM((NCH, CS), jnp.int32),
            pltpu.VMEM((HH, NCH, CS), jnp.int32),
            pltpu.VMEM((NCH, CS), jnp.int32),
            pltpu.VMEM((NBUF, CS, DP), jnp.float32),
            pltpu.VMEM((D, CS), jnp.int32),
            pltpu.VMEM((D, CS), jnp.float32),
            pltpu.VMEM((R,), jnp.float32),
            pltpu.VMEM_SHARED((NS * R, DP), jnp.float32),
            pltpu.SemaphoreType.DMA,
            pltpu.SemaphoreType.DMA,
        ],
        compiler_params=pltpu.CompilerParams(
            needs_layout_passes=False, use_tc_tiling_on_sc=False
        ),
    )
    return run(items_f, ftab_p, iids, fids)


# speedup vs baseline: 6.5258x; 2.4434x over previous
import jax
import jax.numpy as jnp
from jax import lax
from jax.experimental import pallas as pl
from jax.experimental.pallas import tpu as pltpu
from jax.experimental.pallas import tpu_sc as plsc

B = 16384
D = 65
DP = 128
V = 1000000
H = 50
HH = 25
NC = 2
NS = 16
NW = NC * NS
R = B // NW
CS = 128
NCH = R // CS
NBUF = 2
IDEPTH = 8


def _fedrec_body(items_hbm, ftab_hbm, iids_hbm, fids_hbm, out_hbm,
                 iidx_v, fidx_v, ramp_v, feat_v, out_v,
                 acc_sh, sem, sem2):
    c = lax.axis_index("c")
    s = lax.axis_index("s")
    wid = s * NC + c
    blk = wid * NCH
    iota = lax.iota(jnp.int32, 16)

    pltpu.sync_copy(iids_hbm.at[pl.ds(blk, NCH)], iidx_v)

    for sc in range(NCH):
        for k in range(CS // 16):
            ramp_v[sc, pl.ds(k * 16, 16)] = (
                s * R + sc * CS + k * 16 + iota
            )

    zbuf = feat_v.at[0]

    def zero_row(r, carry):
        z16 = jnp.zeros((16,), jnp.float32)
        for j in range(DP // 16):
            zbuf[r, pl.ds(j * 16, 16)] = z16
        return carry

    lax.fori_loop(0, CS, zero_row, 0)
    for sc in range(NCH):
        pltpu.sync_copy(zbuf, acc_sh.at[pl.ds(s * R + sc * CS, CS), :])

    def pool_step(h, carry):
        for sc0 in range(0, NCH, NBUF):
            g = [
                pltpu.async_copy(
                    ftab_hbm.at[fidx_v.at[h, sc0 + i]],
                    feat_v.at[i],
                    sem,
                )
                for i in range(NBUF)
            ]
            for d in g:
                d.wait()
            sca = [
                pltpu.async_copy(
                    feat_v.at[i],
                    acc_sh.at[ramp_v.at[sc0 + i]],
                    sem,
                    add=True,
                )
                for i in range(NBUF)
            ]
            for d in sca:
                d.wait()
        return carry

    for hh in range(H // HH):
        pltpu.sync_copy(
            fids_hbm.at[pl.ds(hh * HH, HH), pl.ds(blk, NCH), :], fidx_v
        )
        lax.fori_loop(0, HH, pool_step, 0)

    item_c = feat_v.at[0]
    pref_c = feat_v.at[1]
    for sc in range(NCH):
        gi = pltpu.async_copy(items_hbm.at[iidx_v.at[sc]], item_c, sem2)
        pltpu.sync_copy(acc_sh.at[pl.ds(s * R + sc * CS, CS), :], pref_c)
        gi.wait()

        def grp_body(g, carry):
            rows16 = g * 16 + iota

            def d_body(d, acc):
                dv = jnp.full((16,), 0, jnp.int32) + d
                a = plsc.load_gather(item_c, [rows16, dv])
                b = plsc.load_gather(pref_c, [rows16, dv])
                return acc + a * b

            acc = lax.fori_loop(0, D, d_body, jnp.zeros((16,), jnp.float32))
            out_v[pl.ds(sc * CS + g * 16, 16)] = acc
            return carry

        lax.fori_loop(0, CS // 16, grp_body, 0)

    pltpu.sync_copy(out_v, out_hbm.at[pl.ds(wid * R, R)])


@jax.jit
def kernel(items_emb, feature_emb, item_ids, feature_ids):
    eye = jnp.eye(D, DP, dtype=jnp.float32)
    items_p = jax.lax.dot(
        items_emb, eye, precision=jax.lax.Precision.HIGHEST
    )
    ftab_p = jax.lax.dot(
        feature_emb, eye, precision=jax.lax.Precision.HIGHEST
    )
    iids = item_ids.astype(jnp.int32).reshape(B // CS, CS)
    fids = feature_ids.astype(jnp.int32).T.reshape(H, B // CS, CS)
    mesh = plsc.VectorSubcoreMesh(core_axis_name="c", subcore_axis_name="s")
    run = pl.kernel(
        _fedrec_body,
        out_type=jax.ShapeDtypeStruct((B,), jnp.float32),
        mesh=mesh,
        scratch_types=[
            pltpu.VMEM((NCH, CS), jnp.int32),
            pltpu.VMEM((HH, NCH, CS), jnp.int32),
            pltpu.VMEM((NCH, CS), jnp.int32),
            pltpu.VMEM((NBUF, CS, DP), jnp.float32),
            pltpu.VMEM((R,), jnp.float32),
            pltpu.VMEM_SHARED((NS * R, DP), jnp.float32),
            pltpu.SemaphoreType.DMA,
            pltpu.SemaphoreType.DMA,
        ],
        compiler_params=pltpu.CompilerParams(
            needs_layout_passes=False, use_tc_tiling_on_sc=False
        ),
    )
    return run(items_p, ftab_p, iids, fids)


# speedup vs baseline: 9.9538x; 1.5253x over previous
import jax
import jax.numpy as jnp
from jax import lax
from jax.experimental import pallas as pl
from jax.experimental.pallas import tpu as pltpu
from jax.experimental.pallas import tpu_sc as plsc

B = 16384
D = 65
DP = 128
H = 50
HH = 25
NC = 2
NS = 16
NW = NC * NS
R = B // NW
CS = 128
NCH = R // CS
NBUF = 2


def _pool_body(ftab_hbm, fids_hbm, pref_hbm,
               fidx_v, ramp_v, feat_v, acc_sh, sem):
    c = lax.axis_index("c")
    s = lax.axis_index("s")
    wid = s * NC + c
    blk = wid * NCH
    iota = lax.iota(jnp.int32, 16)

    for sc in range(NCH):
        for k in range(CS // 16):
            ramp_v[sc, pl.ds(k * 16, 16)] = (
                s * R + sc * CS + k * 16 + iota
            )

    zbuf = feat_v.at[0]

    def zero_row(r, carry):
        z16 = jnp.zeros((16,), jnp.float32)
        for j in range(DP // 16):
            zbuf[r, pl.ds(j * 16, 16)] = z16
        return carry

    lax.fori_loop(0, CS, zero_row, 0)
    for sc in range(NCH):
        pltpu.sync_copy(zbuf, acc_sh.at[pl.ds(s * R + sc * CS, CS), :])

    def pool_step(h, carry):
        for sc0 in range(0, NCH, NBUF):
            g = [
                pltpu.async_copy(
                    ftab_hbm.at[fidx_v.at[h, sc0 + i]],
                    feat_v.at[i],
                    sem,
                )
                for i in range(NBUF)
            ]
            for d in g:
                d.wait()
            sca = [
                pltpu.async_copy(
                    feat_v.at[i],
                    acc_sh.at[ramp_v.at[sc0 + i]],
                    sem,
                    add=True,
                )
                for i in range(NBUF)
            ]
            for d in sca:
                d.wait()
        return carry

    for hh in range(H // HH):
        pltpu.sync_copy(
            fids_hbm.at[pl.ds(hh * HH, HH), pl.ds(blk, NCH), :], fidx_v
        )
        lax.fori_loop(0, HH, pool_step, 0)

    pltpu.sync_copy(
        acc_sh.at[pl.ds(s * R, R), :], pref_hbm.at[pl.ds(wid * R, R), :]
    )


def _dot_body(items_hbm, iids_hbm, pref_hbm, out_hbm,
              iidx_v, feat_v, out_v, sem, sem2):
    c = lax.axis_index("c")
    s = lax.axis_index("s")
    wid = s * NC + c
    blk = wid * NCH
    iota = lax.iota(jnp.int32, 16)

    pltpu.sync_copy(iids_hbm.at[pl.ds(blk, NCH)], iidx_v)

    item_c = feat_v.at[0]
    pref_c = feat_v.at[1]
    for sc in range(NCH):
        gi = pltpu.async_copy(items_hbm.at[iidx_v.at[sc]], item_c, sem2)
        pltpu.sync_copy(
            pref_hbm.at[pl.ds(wid * R + sc * CS, CS), :], pref_c
        )
        gi.wait()

        def grp_body(g, carry):
            rows16 = g * 16 + iota

            def d_body(d, acc):
                dv = jnp.full((16,), 0, jnp.int32) + d
                a = plsc.load_gather(item_c, [rows16, dv])
                b = plsc.load_gather(pref_c, [rows16, dv])
                return acc + a * b

            acc = lax.fori_loop(0, D, d_body, jnp.zeros((16,), jnp.float32))
            out_v[pl.ds(sc * CS + g * 16, 16)] = acc
            return carry

        lax.fori_loop(0, CS // 16, grp_body, 0)

    pltpu.sync_copy(out_v, out_hbm.at[pl.ds(wid * R, R)])


@jax.jit
def kernel(items_emb, feature_emb, item_ids, feature_ids):
    eye = jnp.eye(D, DP, dtype=jnp.float32)
    items_p = jax.lax.dot(
        items_emb, eye, precision=jax.lax.Precision.HIGHEST
    )
    ftab_p = jax.lax.dot(
        feature_emb, eye, precision=jax.lax.Precision.HIGHEST
    )
    iids = item_ids.astype(jnp.int32).reshape(B // CS, CS)
    fids = feature_ids.astype(jnp.int32).T.reshape(H, B // CS, CS)
    mesh = plsc.VectorSubcoreMesh(core_axis_name="c", subcore_axis_name="s")

    pool = pl.kernel(
        _pool_body,
        out_type=jax.ShapeDtypeStruct((B, DP), jnp.float32),
        mesh=mesh,
        scratch_types=[
            pltpu.VMEM((HH, NCH, CS), jnp.int32),
            pltpu.VMEM((NCH, CS), jnp.int32),
            pltpu.VMEM((NBUF, CS, DP), jnp.float32),
            pltpu.VMEM_SHARED((NS * R, DP), jnp.float32),
            pltpu.SemaphoreType.DMA,
        ],
        compiler_params=pltpu.CompilerParams(
            needs_layout_passes=False, use_tc_tiling_on_sc=False
        ),
    )
    pref = pool(ftab_p, fids)

    dot = pl.kernel(
        _dot_body,
        out_type=jax.ShapeDtypeStruct((B,), jnp.float32),
        mesh=mesh,
        scratch_types=[
            pltpu.VMEM((NCH, CS), jnp.int32),
            pltpu.VMEM((2, CS, DP), jnp.float32),
            pltpu.VMEM((R,), jnp.float32),
            pltpu.SemaphoreType.DMA,
            pltpu.SemaphoreType.DMA,
        ],
        compiler_params=pltpu.CompilerParams(
            needs_layout_passes=False, use_tc_tiling_on_sc=False
        ),
    )
    return dot(items_p, iids, pref)
